# baseline (speedup 1.0000x reference)
import jax
import jax.numpy as jnp
from jax.experimental import pallas as pl
from jax.experimental.pallas import tpu as pltpu

_NL = 128
_DEPTH = 4


def _body(TN, k_static):
    def body(d_ref, dt_ref, dxr_ref, dxc_ref,
             c_ref, ct_ref, cxr_ref, cxc_ref,
             wt_ref, b_ref, o_ref, idx0_ref, idx1_ref):
        N = d_ref.shape[2]
        G = N // _NL

        iota_full = jax.lax.broadcasted_iota(jnp.int32, (TN, N), 1)
        iota_nl = jax.lax.broadcasted_iota(jnp.int32, (TN, _NL), 1)
        lane_k = jax.lax.broadcasted_iota(jnp.int32, (TN, k_static), 1)
        neg_inf = jnp.float32(-jnp.inf)

        def make_pd(x_ref, xt_ref, xr_ref, xc_ref):
            x = x_ref[0]
            rows = xt_ref[0]
            g = jax.lax.dot_general(
                rows, x, (((1,), (0,)), ((), ())),
                precision=jax.lax.Precision.DEFAULT,
                preferred_element_type=jnp.float32)
            inner = -2.0 * g
            return (-xr_ref[0]) - inner - xc_ref[0]

        pds = [make_pd(d_ref, dt_ref, dxr_ref, dxc_ref),
               make_pd(c_ref, ct_ref, cxr_ref, cxc_ref)]
        idx_refs = [idx0_ref, idx1_ref]

        def ce(a, b):
            c = a[0] >= b[0]
            return ((jnp.where(c, a[0], b[0]), jnp.where(c, a[1], b[1])),
                    (jnp.where(c, b[0], a[0]), jnp.where(c, b[1], a[1])))

        def ce_max(a, b):
            c = a[0] >= b[0]
            return (jnp.where(c, a[0], b[0]), jnp.where(c, a[1], b[1]))

        def merge22(A, B):
            hi, x = ce(A[0], B[0])
            y, lo = ce(A[1], B[1])
            mid_hi, mid_lo = ce(x, y)
            return [hi, mid_hi, mid_lo, lo]

        def merge44(A, B, sort_result):
            c = [ce_max(A[i], B[3 - i]) for i in range(4)]
            if not sort_result:
                return c
            c0, c2 = ce(c[0], c[2])
            c1, c3 = ce(c[1], c[3])
            c0, c1 = ce(c0, c1)
            c2, c3 = ce(c2, c3)
            return [c0, c1, c2, c3]

        neg_tile = (jnp.full((TN, _NL), neg_inf, jnp.float32),
                    jnp.zeros((TN, _NL), jnp.int32))

        def build_top4(tiles):
            sing = list(tiles)
            if len(sing) % 2:
                sing.append(neg_tile)
            pairs = [ce(sing[2 * p], sing[2 * p + 1])
                     for p in range(len(sing) // 2)]
            if len(pairs) % 2:
                pairs.append((neg_tile, neg_tile))
            quads = [merge22(pairs[2 * q], pairs[2 * q + 1])
                     for q in range(len(pairs) // 2)]
            while len(quads) > 1:
                if len(quads) % 2:
                    quads.append([neg_tile] * 4)
                quads = [merge44(quads[2 * r], quads[2 * r + 1],
                                 len(quads) > 2)
                         for r in range(len(quads) // 2)]
            return quads[0]

        top = [build_top4([(pds[s][:, t * _NL:(t + 1) * _NL],
                            iota_nl + t * _NL) for t in range(G)])
               for s in range(2)]
        ms = [[top[s][i][0] for i in range(_DEPTH)] for s in range(2)]
        js = [[top[s][i][1] for i in range(_DEPTH)] for s in range(2)]

        cands = [jnp.concatenate(ms[s], axis=1) for s in range(2)]
        cidxs = [jnp.concatenate(js[s], axis=1) for s in range(2)]

        out_idxs = [jnp.zeros((TN, k_static), jnp.int32) for _ in range(2)]
        mxs = [None, None]
        sels = [None, None]

        def top2(v):
            NCW = v.shape[1]
            a1, b1 = v[:, :NCW // 2], v[:, NCW // 2:]
            m1 = jnp.maximum(a1, b1)
            m2 = jnp.minimum(a1, b1)
            while m1.shape[1] > 1:
                w = m1.shape[1] // 2
                a1, b1 = m1[:, :w], m1[:, w:]
                a2, b2 = m2[:, :w], m2[:, w:]
                m1 = jnp.maximum(a1, b1)
                m2 = jnp.maximum(jnp.minimum(a1, b1), jnp.maximum(a2, b2))
            return m1, m2

        for i in range(0, k_static - 1, 2):
            for s in range(2):
                m1, m2 = top2(cands[s])
                sel1 = jnp.min(jnp.where(cands[s] == m1, cidxs[s], N),
                               axis=1, keepdims=True)
                mask2 = (cands[s] == m2) & ((m2 < m1) | (cidxs[s] > sel1))
                sel2 = jnp.min(jnp.where(mask2, cidxs[s], N),
                               axis=1, keepdims=True)
                out_idxs[s] = jnp.where(lane_k == i, sel1, out_idxs[s])
                out_idxs[s] = jnp.where(lane_k == i + 1, sel2, out_idxs[s])
                if i + 2 < k_static:
                    cands[s] = jnp.where((cidxs[s] == sel1) |
                                         (cidxs[s] == sel2),
                                         neg_inf, cands[s])
                mxs[s], sels[s] = m2, sel2

        if k_static % 2:
            for s in range(2):
                mx = jnp.max(cands[s], axis=1, keepdims=True)
                sel = jnp.min(jnp.where(cands[s] == mx, cidxs[s], N),
                              axis=1, keepdims=True)
                out_idxs[s] = jnp.where(lane_k == k_static - 1, sel,
                                        out_idxs[s])
                mxs[s], sels[s] = mx, sel

        for s in range(2):
            idx_refs[s][...] = out_idxs[s]
            above = ((pds[s] > mxs[s]) |
                     ((pds[s] == mxs[s]) & (iota_full < sels[s])))
            cnt = jnp.sum(above.astype(jnp.int32), axis=1, keepdims=True)
            ok = jnp.sum(jnp.where(cnt == k_static - 1, 0, 1)) == 0

            @pl.when(jnp.logical_not(ok))
            def _fallback(s=s):
                srcs = [(d_ref, dt_ref, dxr_ref, dxc_ref),
                        (c_ref, ct_ref, cxr_ref, cxc_ref)]
                fvals = make_pd(*srcs[s])
                fidx = jnp.zeros((TN, k_static), jnp.int32)
                for i in range(k_static):
                    fm = jnp.max(fvals, axis=1, keepdims=True)
                    fj = jnp.min(jnp.where(fvals == fm, iota_full, N),
                                 axis=1, keepdims=True)
                    fidx = jnp.where(lane_k == i, fj, fidx)
                    if i + 1 < k_static:
                        fvals = jnp.where(iota_full == fj, neg_inf, fvals)
                idx_refs[s][...] = fidx

        fuse = (idx_refs[0][...] + idx_refs[1][...]).astype(jnp.float32)
        o = jax.lax.dot_general(
            fuse, wt_ref[...], (((1,), (0,)), ((), ())),
            precision=jax.lax.Precision.DEFAULT,
            preferred_element_type=jnp.float32) + b_ref[...]
        o_ref[0] = o

    return body


def kernel(k, depth, color, W, b):
    B, C, N = depth.shape
    out_hid, k_static = W.shape
    TN = min(256, N)

    xx_d = jnp.sum(depth ** 2, axis=1, keepdims=True)
    xx_c = jnp.sum(color ** 2, axis=1, keepdims=True)
    dxr = jnp.transpose(xx_d, (0, 2, 1))
    cxr = jnp.transpose(xx_c, (0, 2, 1))
    depth_t = jnp.transpose(depth, (0, 2, 1))
    color_t = jnp.transpose(color, (0, 2, 1))
    Wt = jnp.transpose(W)
    b2 = jnp.reshape(b, (1, out_hid))

    grid = (B, N // TN)
    out = pl.pallas_call(
        _body(TN, k_static),
        grid=grid,
        in_specs=[
            pl.BlockSpec((1, C, N), lambda bi, ti: (bi, 0, 0)),
            pl.BlockSpec((1, TN, C), lambda bi, ti: (bi, ti, 0)),
            pl.BlockSpec((1, TN, 1), lambda bi, ti: (bi, ti, 0)),
            pl.BlockSpec((1, 1, N), lambda bi, ti: (bi, 0, 0)),
            pl.BlockSpec((1, C, N), lambda bi, ti: (bi, 0, 0)),
            pl.BlockSpec((1, TN, C), lambda bi, ti: (bi, ti, 0)),
            pl.BlockSpec((1, TN, 1), lambda bi, ti: (bi, ti, 0)),
            pl.BlockSpec((1, 1, N), lambda bi, ti: (bi, 0, 0)),
            pl.BlockSpec((k_static, out_hid), lambda bi, ti: (0, 0)),
            pl.BlockSpec((1, out_hid), lambda bi, ti: (0, 0)),
        ],
        out_specs=pl.BlockSpec((1, TN, out_hid), lambda bi, ti: (bi, ti, 0)),
        out_shape=jax.ShapeDtypeStruct((B, N, out_hid), jnp.float32),
        scratch_shapes=[
            pltpu.VMEM((TN, k_static), jnp.int32),
            pltpu.VMEM((TN, k_static), jnp.int32),
        ],
    )(depth, depth_t, dxr, xx_d, color, color_t, cxr, xx_c, Wt, b2)
    return out


# speedup vs baseline: 3.7613x; 3.7613x over previous
import jax
import jax.numpy as jnp
from jax.experimental import pallas as pl
from jax.experimental.pallas import tpu as pltpu

_NL = 128
_DEPTH = 4


def _body(TN, k_static):
    def body(d_ref, dt_ref, dxr_ref, dxc_ref,
             c_ref, ct_ref, cxr_ref, cxc_ref,
             wt_ref, b_ref, o_ref, idx0_ref, idx1_ref):
        N = d_ref.shape[2]
        G = N // _NL

        iota_full = jax.lax.broadcasted_iota(jnp.int32, (TN, N), 1)
        iota_nl = jax.lax.broadcasted_iota(jnp.int32, (TN, _NL), 1)
        lane_k = jax.lax.broadcasted_iota(jnp.int32, (TN, k_static), 1)
        neg_inf = jnp.float32(-jnp.inf)

        def make_pd(x_ref, xt_ref, xr_ref, xc_ref):
            x = x_ref[0]
            rows = xt_ref[0]
            g = jax.lax.dot_general(
                rows, x, (((1,), (0,)), ((), ())),
                precision=jax.lax.Precision.DEFAULT,
                preferred_element_type=jnp.float32)
            inner = -2.0 * g
            return (-xr_ref[0]) - inner - xc_ref[0]

        pds = [make_pd(d_ref, dt_ref, dxr_ref, dxc_ref),
               make_pd(c_ref, ct_ref, cxr_ref, cxc_ref)]
        idx_refs = [idx0_ref, idx1_ref]

        def ce(a, b):
            c = a[0] >= b[0]
            return ((jnp.where(c, a[0], b[0]), jnp.where(c, a[1], b[1])),
                    (jnp.where(c, b[0], a[0]), jnp.where(c, b[1], a[1])))

        def ce_max(a, b):
            c = a[0] >= b[0]
            return (jnp.where(c, a[0], b[0]), jnp.where(c, a[1], b[1]))

        def merge22(A, B):
            hi, x = ce(A[0], B[0])
            y, lo = ce(A[1], B[1])
            mid_hi, mid_lo = ce(x, y)
            return [hi, mid_hi, mid_lo, lo]

        def merge44(A, B, sort_result):
            c = [ce_max(A[i], B[3 - i]) for i in range(4)]
            if not sort_result:
                return c
            c0, c2 = ce(c[0], c[2])
            c1, c3 = ce(c[1], c[3])
            c0, c1 = ce(c0, c1)
            c2, c3 = ce(c2, c3)
            return [c0, c1, c2, c3]

        neg_tile = (jnp.full((TN, _NL), neg_inf, jnp.float32),
                    jnp.zeros((TN, _NL), jnp.int32))

        def build_top4(tiles):
            sing = list(tiles)
            if len(sing) % 2:
                sing.append(neg_tile)
            pairs = [ce(sing[2 * p], sing[2 * p + 1])
                     for p in range(len(sing) // 2)]
            if len(pairs) % 2:
                pairs.append((neg_tile, neg_tile))
            quads = [merge22(pairs[2 * q], pairs[2 * q + 1])
                     for q in range(len(pairs) // 2)]
            while len(quads) > 1:
                if len(quads) % 2:
                    quads.append([neg_tile] * 4)
                quads = [merge44(quads[2 * r], quads[2 * r + 1],
                                 len(quads) > 2)
                         for r in range(len(quads) // 2)]
            return quads[0]

        top = [build_top4([(pds[s][:, t * _NL:(t + 1) * _NL],
                            iota_nl + t * _NL) for t in range(G)])
               for s in range(2)]
        ms = [[top[s][i][0] for i in range(_DEPTH)] for s in range(2)]
        js = [[top[s][i][1] for i in range(_DEPTH)] for s in range(2)]

        cands = [jnp.concatenate(ms[s], axis=1) for s in range(2)]
        cidxs = [jnp.concatenate(js[s], axis=1) for s in range(2)]

        out_idxs = [jnp.zeros((TN, k_static), jnp.int32) for _ in range(2)]
        mxs = [None, None]
        sels = [None, None]
        for i in range(k_static):
            for s in range(2):
                mx = jnp.max(cands[s], axis=1, keepdims=True)
                sel = jnp.min(jnp.where(cands[s] == mx, cidxs[s], N),
                              axis=1, keepdims=True)
                out_idxs[s] = jnp.where(lane_k == i, sel, out_idxs[s])
                if i + 1 < k_static:
                    cands[s] = jnp.where(cidxs[s] == sel, neg_inf, cands[s])
                mxs[s], sels[s] = mx, sel

        for s in range(2):
            idx_refs[s][...] = out_idxs[s]
            above = ((pds[s] > mxs[s]) |
                     ((pds[s] == mxs[s]) & (iota_full < sels[s])))
            cnt = jnp.sum(above.astype(jnp.int32), axis=1, keepdims=True)
            ok = jnp.sum(jnp.where(cnt == k_static - 1, 0, 1)) == 0

            @pl.when(jnp.logical_not(ok))
            def _fallback(s=s):
                srcs = [(d_ref, dt_ref, dxr_ref, dxc_ref),
                        (c_ref, ct_ref, cxr_ref, cxc_ref)]
                fvals = make_pd(*srcs[s])
                fidx = jnp.zeros((TN, k_static), jnp.int32)
                for i in range(k_static):
                    fm = jnp.max(fvals, axis=1, keepdims=True)
                    fj = jnp.min(jnp.where(fvals == fm, iota_full, N),
                                 axis=1, keepdims=True)
                    fidx = jnp.where(lane_k == i, fj, fidx)
                    if i + 1 < k_static:
                        fvals = jnp.where(iota_full == fj, neg_inf, fvals)
                idx_refs[s][...] = fidx

        fuse = (idx_refs[0][...] + idx_refs[1][...]).astype(jnp.float32)
        o = jax.lax.dot_general(
            fuse, wt_ref[...], (((1,), (0,)), ((), ())),
            precision=jax.lax.Precision.DEFAULT,
            preferred_element_type=jnp.float32) + b_ref[...]
        o_ref[0] = o

    return body


def kernel(k, depth, color, W, b):
    B, C, N = depth.shape
    out_hid, k_static = W.shape
    TN = min(256, N)

    xx_d = jnp.sum(depth ** 2, axis=1, keepdims=True)
    xx_c = jnp.sum(color ** 2, axis=1, keepdims=True)
    dxr = jnp.transpose(xx_d, (0, 2, 1))
    cxr = jnp.transpose(xx_c, (0, 2, 1))
    depth_t = jnp.transpose(depth, (0, 2, 1))
    color_t = jnp.transpose(color, (0, 2, 1))
    Wt = jnp.transpose(W)
    b2 = jnp.reshape(b, (1, out_hid))

    grid = (B, N // TN)
    out = pl.pallas_call(
        _body(TN, k_static),
        grid=grid,
        in_specs=[
            pl.BlockSpec((1, C, N), lambda bi, ti: (bi, 0, 0)),
            pl.BlockSpec((1, TN, C), lambda bi, ti: (bi, ti, 0)),
            pl.BlockSpec((1, TN, 1), lambda bi, ti: (bi, ti, 0)),
            pl.BlockSpec((1, 1, N), lambda bi, ti: (bi, 0, 0)),
            pl.BlockSpec((1, C, N), lambda bi, ti: (bi, 0, 0)),
            pl.BlockSpec((1, TN, C), lambda bi, ti: (bi, ti, 0)),
            pl.BlockSpec((1, TN, 1), lambda bi, ti: (bi, ti, 0)),
            pl.BlockSpec((1, 1, N), lambda bi, ti: (bi, 0, 0)),
            pl.BlockSpec((k_static, out_hid), lambda bi, ti: (0, 0)),
            pl.BlockSpec((1, out_hid), lambda bi, ti: (0, 0)),
        ],
        out_specs=pl.BlockSpec((1, TN, out_hid), lambda bi, ti: (bi, ti, 0)),
        out_shape=jax.ShapeDtypeStruct((B, N, out_hid), jnp.float32),
        scratch_shapes=[
            pltpu.VMEM((TN, k_static), jnp.int32),
            pltpu.VMEM((TN, k_static), jnp.int32),
        ],
    )(depth, depth_t, dxr, xx_d, color, color_t, cxr, xx_c, Wt, b2)
    return out
